# Initial kernel scaffold; baseline (speedup 1.0000x reference)
import jax
import jax.numpy as jnp
from jax.experimental import pallas as pl


def kernel(news_graph_embeddings, news_graph, news_graph_mask, user_news_embedding, user_graph, user_category_mask, user_category_indices, topic_node_embedding, K_w, Q_w, Q_b, fa_w, fa_b, sa_K_w, sa_Q_w, sa_Q_b, gW_w, gW_b, gf1_w, gf2_w, gf3_w, gf3_b, ga_w):
    raise NotImplementedError("write your pallas kernel here")


# speedup vs baseline: 1.4821x; 1.4821x over previous
import jax
import jax.numpy as jnp
from jax.experimental import pallas as pl

_D = 256
_H = 50
_CAT = 30
_C = _CAT + 1
_G = _H + _CAT
_L = 2
_INV_SQRT_D = 0.0625


def _mmT(x, w):
    return jax.lax.dot_general(
        x, w, (((1,), (1,)), ((), ())), preferred_element_type=jnp.float32)


def _mm(x, w):
    return jax.lax.dot_general(
        x, w, (((1,), (0,)), ((), ())), preferred_element_type=jnp.float32)


def _fused_kernel(ug_ref, ctx_ref, adj_ref, idx_ref, cmask_ref,
                  gW_w_ref, gW_b_ref, gf1_ref, gf2_ref, gf3_ref, gf3_b_ref,
                  ga_ref, K_w_ref, Q_w_ref, Q_b_ref, fa_w_ref, fa_b_ref,
                  saK_ref, saQ_ref, saQb_ref, out_ref):
    ug = ug_ref[0]
    ctx = ctx_ref[0]
    adj = adj_ref[0]

    for i in range(_L):
        h = _mmT(ug, gW_w_ref[i]) + gW_b_ref[i][None, :]
        K1 = _mmT(ug, gf1_ref[i])
        K2 = _mmT(ug, gf2_ref[i])
        K3 = _mmT(ctx, gf3_ref[i]) + gf3_b_ref[i][None, :]
        s12 = K1[None, :, :] + K2[:, None, :] + K3[0][None, None, :]
        e3 = jax.nn.relu(s12)
        a = jnp.sum(e3 * ga_ref[i][None, None, :], axis=-1)
        e = jnp.where(a >= 0, a, 0.2 * a)
        e = jnp.where(adj == 0, -1e9, e)
        e = e - jnp.max(e, axis=1, keepdims=True)
        p = jnp.exp(e)
        alpha = p / jnp.sum(p, axis=1, keepdims=True)
        ug = jax.nn.relu(_mm(alpha, h)) + ug

    hist = ug[:_H]
    Kh = _mmT(hist, K_w_ref[...])
    Qc = _mmT(ctx, Q_w_ref[...]) + Q_b_ref[...]
    a_r = _mmT(Qc, Kh) * _INV_SQRT_D

    idx = idx_ref[0]
    cat_iota = jax.lax.broadcasted_iota(jnp.int32, (_C, _H), 0)
    M = idx == cat_iota
    neg_inf = jnp.float32(-jnp.inf)
    mxc = jnp.max(jnp.where(M, a_r, neg_inf), axis=1, keepdims=True)
    mxc = jnp.where(mxc == neg_inf, 0.0, mxc)
    m = jnp.sum(jnp.where(M, mxc, 0.0), axis=0, keepdims=True)
    ex = jnp.exp(a_r - m)
    den = jnp.sum(jnp.where(M, ex, 0.0), axis=1, keepdims=True)
    denh = jnp.sum(jnp.where(M, den, 0.0), axis=0, keepdims=True)
    w_sc = jnp.where(M, ex / denh, 0.0)
    topics = _mm(w_sc, hist)
    t2 = jax.nn.relu(_mmT(topics, fa_w_ref[...]) + fa_b_ref[...]) + topics

    Ks = _mmT(t2, saK_ref[...])
    Qs = _mmT(ctx, saQ_ref[...]) + saQb_ref[...]
    s = _mmT(Qs, Ks) * _INV_SQRT_D
    cm = cmask_ref[0]
    s = jnp.where(cm == 0, -1e9, s)
    s = s - jnp.max(s, axis=1, keepdims=True)
    ps = jnp.exp(s)
    al = ps / jnp.sum(ps, axis=1, keepdims=True)
    out_ref[0] = _mm(al, t2)


def kernel(news_graph_embeddings, news_graph, news_graph_mask,
           user_news_embedding, user_graph, user_category_mask,
           user_category_indices, topic_node_embedding,
           K_w, Q_w, Q_b, fa_w, fa_b, sa_K_w, sa_Q_w, sa_Q_b,
           gW_w, gW_b, gf1_w, gf2_w, gf3_w, gf3_b, ga_w):
    B = news_graph_embeddings.shape[0]
    f32 = jnp.float32

    ctx = news_graph_embeddings[:, 0, :]
    topic_nodes = jnp.broadcast_to(topic_node_embedding[None], (B, _CAT, _D))
    ug0 = jnp.concatenate([user_news_embedding, topic_nodes], axis=1)

    ctx3 = ctx.reshape(B, 1, _D)
    idx3 = user_category_indices.reshape(B, 1, _H).astype(jnp.int32)
    cmask3 = user_category_mask.reshape(B, 1, _C).astype(jnp.int32)
    Q_b2 = Q_b.reshape(1, _D)
    fa_b2 = fa_b.reshape(1, _D)
    sa_Q_b2 = sa_Q_b.reshape(1, _D)

    def full(spec_shape):
        return pl.BlockSpec(spec_shape, lambda b: (0,) * len(spec_shape))

    def per_b(spec_shape):
        return pl.BlockSpec(spec_shape, lambda b: (b,) + (0,) * (len(spec_shape) - 1))

    out = pl.pallas_call(
        _fused_kernel,
        grid=(B,),
        in_specs=[
            per_b((1, _G, _D)),
            per_b((1, 1, _D)),
            per_b((1, _G, _G)),
            per_b((1, 1, _H)),
            per_b((1, 1, _C)),
            full((_L, _D, _D)),
            full((_L, _D)),
            full((_L, _D, _D)),
            full((_L, _D, _D)),
            full((_L, _D, _D)),
            full((_L, _D)),
            full((_L, _D)),
            full((_D, _D)),
            full((_D, _D)),
            full((1, _D)),
            full((_D, _D)),
            full((1, _D)),
            full((_D, _D)),
            full((_D, _D)),
            full((1, _D)),
        ],
        out_specs=per_b((1, 1, _D)),
        out_shape=jax.ShapeDtypeStruct((B, 1, _D), f32),
    )(ug0, ctx3, user_graph, idx3, cmask3,
      gW_w, gW_b, gf1_w, gf2_w, gf3_w, gf3_b, ga_w,
      K_w, Q_w, Q_b2, fa_w, fa_b2, sa_K_w, sa_Q_w, sa_Q_b2)

    return (ctx, out.reshape(B, _D))


# speedup vs baseline: 2.2941x; 1.5478x over previous
import jax
import jax.numpy as jnp
from jax.experimental import pallas as pl

_D = 256
_H = 50
_CAT = 30
_C = _CAT + 1
_G = _H + _CAT
_L = 2
_S = 8
_INV_SQRT_D = 0.0625


def _mmT(x, w):
    return jax.lax.dot_general(
        x, w, (((1,), (1,)), ((), ())), preferred_element_type=jnp.float32)


def _bmm(x, y):
    return jax.lax.dot_general(
        x, y, (((2,), (1,)), ((0,), (0,))), preferred_element_type=jnp.float32)


def _bmv(x, v):
    return jax.lax.dot_general(
        x, v, (((2,), (1,)), ((0,), (0,))), preferred_element_type=jnp.float32)


def _fused_kernel(ug_ref, ctx_ref, adj_ref, idx_ref, cmask_ref,
                  gW_w_ref, gW_b_ref, gf1_ref, gf2_ref, gf3_ref, gf3_b_ref,
                  ga_ref, K_w_ref, Q_w_ref, Q_b_ref, fa_w_ref, fa_b_ref,
                  saK_ref, saQ_ref, saQb_ref, out_ref):
    ug = ug_ref[...].reshape(_S * _G, _D)
    ctx = ctx_ref[...].reshape(_S, _D)
    adj = adj_ref[...]

    for i in range(_L):
        h = _mmT(ug, gW_w_ref[i]) + gW_b_ref[i][None, :]
        K1 = _mmT(ug, gf1_ref[i])
        K2 = _mmT(ug, gf2_ref[i])
        K3 = _mmT(ctx, gf3_ref[i]) + gf3_b_ref[i][None, :]
        u = (K1.reshape(_S, _G, _D) + K3[:, None, :])
        e3 = jax.nn.relu(u[:, None, :, :]
                         + K2.reshape(_S, _G, _D)[:, :, None, :])
        a = jnp.sum(e3 * ga_ref[i][None, None, None, :], axis=-1)
        e = jnp.where(a >= 0, a, 0.2 * a)
        e = jnp.where(adj == 0, -1e9, e)
        e = e - jnp.max(e, axis=-1, keepdims=True)
        p = jnp.exp(e)
        alpha = p / jnp.sum(p, axis=-1, keepdims=True)
        comb = _bmm(alpha, h.reshape(_S, _G, _D)).reshape(_S * _G, _D)
        ug = jax.nn.relu(comb) + ug

    ug3 = ug.reshape(_S, _G, _D)
    hist = ug3[:, :_H, :]
    Kh = _mmT(hist.reshape(_S * _H, _D), K_w_ref[...]).reshape(_S, _H, _D)
    Qc = _mmT(ctx, Q_w_ref[...]) + Q_b_ref[...]
    a_r = _bmv(Kh, Qc) * _INV_SQRT_D

    idx = idx_ref[...].reshape(_S, 1, _H)
    cat_iota = jax.lax.broadcasted_iota(jnp.int32, (_S, _C, _H), 1)
    M = idx == cat_iota
    neg_inf = jnp.float32(-jnp.inf)
    a_b = a_r[:, None, :]
    mxc = jnp.max(jnp.where(M, a_b, neg_inf), axis=2, keepdims=True)
    mxc = jnp.where(mxc == neg_inf, 0.0, mxc)
    m = jnp.sum(jnp.where(M, mxc, 0.0), axis=1, keepdims=True)
    ex = jnp.exp(a_b - m)
    den = jnp.sum(jnp.where(M, ex, 0.0), axis=2, keepdims=True)
    denh = jnp.sum(jnp.where(M, den, 0.0), axis=1, keepdims=True)
    w_sc = jnp.where(M, ex / denh, 0.0)
    topics = _bmm(w_sc, hist)
    t2 = jax.nn.relu(
        _mmT(topics.reshape(_S * _C, _D), fa_w_ref[...]) + fa_b_ref[...]
    ).reshape(_S, _C, _D) + topics

    Ks = _mmT(t2.reshape(_S * _C, _D), saK_ref[...]).reshape(_S, _C, _D)
    Qs = _mmT(ctx, saQ_ref[...]) + saQb_ref[...]
    s = _bmv(Ks, Qs) * _INV_SQRT_D
    cm = cmask_ref[...].reshape(_S, _C)
    s = jnp.where(cm == 0, -1e9, s)
    s = s - jnp.max(s, axis=-1, keepdims=True)
    ps = jnp.exp(s)
    al = ps / jnp.sum(ps, axis=-1, keepdims=True)
    out_ref[...] = _bmm(al[:, None, :], t2)


def kernel(news_graph_embeddings, news_graph, news_graph_mask,
           user_news_embedding, user_graph, user_category_mask,
           user_category_indices, topic_node_embedding,
           K_w, Q_w, Q_b, fa_w, fa_b, sa_K_w, sa_Q_w, sa_Q_b,
           gW_w, gW_b, gf1_w, gf2_w, gf3_w, gf3_b, ga_w):
    B = news_graph_embeddings.shape[0]
    f32 = jnp.float32

    ctx = news_graph_embeddings[:, 0, :]
    topic_nodes = jnp.broadcast_to(topic_node_embedding[None], (B, _CAT, _D))
    ug0 = jnp.concatenate([user_news_embedding, topic_nodes], axis=1)

    ctx3 = ctx.reshape(B, 1, _D)
    idx3 = user_category_indices.reshape(B, 1, _H).astype(jnp.int32)
    cmask3 = user_category_mask.reshape(B, 1, _C).astype(jnp.int32)
    Q_b2 = Q_b.reshape(1, _D)
    fa_b2 = fa_b.reshape(1, _D)
    sa_Q_b2 = sa_Q_b.reshape(1, _D)

    def full(spec_shape):
        return pl.BlockSpec(spec_shape, lambda b: (0,) * len(spec_shape))

    def per_b(spec_shape):
        return pl.BlockSpec(spec_shape, lambda b: (b,) + (0,) * (len(spec_shape) - 1))

    out = pl.pallas_call(
        _fused_kernel,
        grid=(B // _S,),
        in_specs=[
            per_b((_S, _G, _D)),
            per_b((_S, 1, _D)),
            per_b((_S, _G, _G)),
            per_b((_S, 1, _H)),
            per_b((_S, 1, _C)),
            full((_L, _D, _D)),
            full((_L, _D)),
            full((_L, _D, _D)),
            full((_L, _D, _D)),
            full((_L, _D, _D)),
            full((_L, _D)),
            full((_L, _D)),
            full((_D, _D)),
            full((_D, _D)),
            full((1, _D)),
            full((_D, _D)),
            full((1, _D)),
            full((_D, _D)),
            full((_D, _D)),
            full((1, _D)),
        ],
        out_specs=per_b((_S, 1, _D)),
        out_shape=jax.ShapeDtypeStruct((B, 1, _D), f32),
    )(ug0, ctx3, user_graph, idx3, cmask3,
      gW_w, gW_b, gf1_w, gf2_w, gf3_w, gf3_b, ga_w,
      K_w, Q_w, Q_b2, fa_w, fa_b2, sa_K_w, sa_Q_w, sa_Q_b2)

    return (ctx, out.reshape(B, _D))


# speedup vs baseline: 2.3806x; 1.0377x over previous
import jax
import jax.numpy as jnp
from jax.experimental import pallas as pl

_D = 256
_H = 50
_CAT = 30
_C = _CAT + 1
_G = _H + _CAT
_L = 2
_S = 8
_INV_SQRT_D = 0.0625


def _mmT(x, w):
    return jax.lax.dot_general(
        x, w, (((1,), (1,)), ((), ())), preferred_element_type=jnp.float32)


def _bmm(x, y):
    return jax.lax.dot_general(
        x, y, (((2,), (1,)), ((0,), (0,))), preferred_element_type=jnp.float32)


def _bmv(x, v):
    return jax.lax.dot_general(
        x, v, (((2,), (1,)), ((0,), (0,))), preferred_element_type=jnp.float32)


def _fused_kernel(ug_ref, ctx_ref, adj_ref, idx_ref, cmask_ref,
                  gW_w_ref, gW_b_ref, gf1_ref, gf2_ref, gf3_ref, gf3_b_ref,
                  ga_ref, K_w_ref, Q_w_ref, Q_b_ref, fa_w_ref, fa_b_ref,
                  saK_ref, saQ_ref, saQb_ref, out_ref):
    ug = ug_ref[...].reshape(_S * _G, _D)
    ctx = ctx_ref[...].reshape(_S, _D)
    adj = adj_ref[...]

    for i in range(_L):
        h = _mmT(ug, gW_w_ref[i]) + gW_b_ref[i][None, :]
        K1 = _mmT(ug, gf1_ref[i])
        K2 = _mmT(ug, gf2_ref[i])
        K3 = _mmT(ctx, gf3_ref[i]) + gf3_b_ref[i][None, :]
        bf16 = jnp.bfloat16
        u = (K1.reshape(_S, _G, _D) + K3[:, None, :]).astype(bf16)
        K2b = K2.reshape(_S, _G, _D).astype(bf16)
        gab = ga_ref[i].astype(bf16)
        e3 = jax.nn.relu(u[:, None, :, :] + K2b[:, :, None, :])
        a = jnp.sum(e3 * gab[None, None, None, :], axis=-1,
                    dtype=jnp.float32)
        e = jnp.where(a >= 0, a, 0.2 * a)
        e = jnp.where(adj == 0, -1e9, e)
        e = e - jnp.max(e, axis=-1, keepdims=True)
        p = jnp.exp(e)
        alpha = p / jnp.sum(p, axis=-1, keepdims=True)
        comb = _bmm(alpha, h.reshape(_S, _G, _D)).reshape(_S * _G, _D)
        ug = jax.nn.relu(comb) + ug

    ug3 = ug.reshape(_S, _G, _D)
    hist = ug3[:, :_H, :]
    Kh = _mmT(hist.reshape(_S * _H, _D), K_w_ref[...]).reshape(_S, _H, _D)
    Qc = _mmT(ctx, Q_w_ref[...]) + Q_b_ref[...]
    a_r = _bmv(Kh, Qc) * _INV_SQRT_D

    idx = idx_ref[...].reshape(_S, 1, _H)
    cat_iota = jax.lax.broadcasted_iota(jnp.int32, (_S, _C, _H), 1)
    M = idx == cat_iota
    neg_inf = jnp.float32(-jnp.inf)
    a_b = a_r[:, None, :]
    mxc = jnp.max(jnp.where(M, a_b, neg_inf), axis=2, keepdims=True)
    mxc = jnp.where(mxc == neg_inf, 0.0, mxc)
    m = jnp.sum(jnp.where(M, mxc, 0.0), axis=1, keepdims=True)
    ex = jnp.exp(a_b - m)
    den = jnp.sum(jnp.where(M, ex, 0.0), axis=2, keepdims=True)
    denh = jnp.sum(jnp.where(M, den, 0.0), axis=1, keepdims=True)
    w_sc = jnp.where(M, ex / denh, 0.0)
    topics = _bmm(w_sc, hist)
    t2 = jax.nn.relu(
        _mmT(topics.reshape(_S * _C, _D), fa_w_ref[...]) + fa_b_ref[...]
    ).reshape(_S, _C, _D) + topics

    Ks = _mmT(t2.reshape(_S * _C, _D), saK_ref[...]).reshape(_S, _C, _D)
    Qs = _mmT(ctx, saQ_ref[...]) + saQb_ref[...]
    s = _bmv(Ks, Qs) * _INV_SQRT_D
    cm = cmask_ref[...].reshape(_S, _C)
    s = jnp.where(cm == 0, -1e9, s)
    s = s - jnp.max(s, axis=-1, keepdims=True)
    ps = jnp.exp(s)
    al = ps / jnp.sum(ps, axis=-1, keepdims=True)
    out_ref[...] = _bmm(al[:, None, :], t2)


def kernel(news_graph_embeddings, news_graph, news_graph_mask,
           user_news_embedding, user_graph, user_category_mask,
           user_category_indices, topic_node_embedding,
           K_w, Q_w, Q_b, fa_w, fa_b, sa_K_w, sa_Q_w, sa_Q_b,
           gW_w, gW_b, gf1_w, gf2_w, gf3_w, gf3_b, ga_w):
    B = news_graph_embeddings.shape[0]
    f32 = jnp.float32

    ctx = news_graph_embeddings[:, 0, :]
    topic_nodes = jnp.broadcast_to(topic_node_embedding[None], (B, _CAT, _D))
    ug0 = jnp.concatenate([user_news_embedding, topic_nodes], axis=1)

    ctx3 = ctx.reshape(B, 1, _D)
    idx3 = user_category_indices.reshape(B, 1, _H).astype(jnp.int32)
    cmask3 = user_category_mask.reshape(B, 1, _C).astype(jnp.int32)
    Q_b2 = Q_b.reshape(1, _D)
    fa_b2 = fa_b.reshape(1, _D)
    sa_Q_b2 = sa_Q_b.reshape(1, _D)

    def full(spec_shape):
        return pl.BlockSpec(spec_shape, lambda b: (0,) * len(spec_shape))

    def per_b(spec_shape):
        return pl.BlockSpec(spec_shape, lambda b: (b,) + (0,) * (len(spec_shape) - 1))

    out = pl.pallas_call(
        _fused_kernel,
        grid=(B // _S,),
        in_specs=[
            per_b((_S, _G, _D)),
            per_b((_S, 1, _D)),
            per_b((_S, _G, _G)),
            per_b((_S, 1, _H)),
            per_b((_S, 1, _C)),
            full((_L, _D, _D)),
            full((_L, _D)),
            full((_L, _D, _D)),
            full((_L, _D, _D)),
            full((_L, _D, _D)),
            full((_L, _D)),
            full((_L, _D)),
            full((_D, _D)),
            full((_D, _D)),
            full((1, _D)),
            full((_D, _D)),
            full((1, _D)),
            full((_D, _D)),
            full((_D, _D)),
            full((1, _D)),
        ],
        out_specs=per_b((_S, 1, _D)),
        out_shape=jax.ShapeDtypeStruct((B, 1, _D), f32),
    )(ug0, ctx3, user_graph, idx3, cmask3,
      gW_w, gW_b, gf1_w, gf2_w, gf3_w, gf3_b, ga_w,
      K_w, Q_w, Q_b2, fa_w, fa_b2, sa_K_w, sa_Q_w, sa_Q_b2)

    return (ctx, out.reshape(B, _D))


# speedup vs baseline: 2.9315x; 1.2314x over previous
import jax
import jax.numpy as jnp
from jax.experimental import pallas as pl

_D = 256
_H = 50
_CAT = 30
_C = _CAT + 1
_G = _H + _CAT
_L = 2
_S = 8
_INV_SQRT_D = 0.0625


def _mmT(x, w):
    return jax.lax.dot_general(
        x, w, (((1,), (1,)), ((), ())), preferred_element_type=jnp.float32)


def _bmm(x, y):
    return jax.lax.dot_general(
        x, y, (((2,), (1,)), ((0,), (0,))), preferred_element_type=jnp.float32)


def _bmv(x, v):
    return jax.lax.dot_general(
        x, v, (((2,), (1,)), ((0,), (0,))), preferred_element_type=jnp.float32)


def _fused_kernel(ug_ref, ctx_ref, adj_ref, idx_ref, cmask_ref,
                  gW_w_ref, gW_b_ref, gf1_ref, gf2_ref, gf3_ref, gf3_b_ref,
                  ga_ref, K_w_ref, Q_w_ref, Q_b_ref, fa_w_ref, fa_b_ref,
                  saK_ref, saQ_ref, saQb_ref, out_ref):
    ug = ug_ref[...].reshape(_S * _G, _D)
    ctx = ctx_ref[...].reshape(_S, _D)
    adj = adj_ref[...]

    for i in range(_L):
        h = _mmT(ug, gW_w_ref[i]) + gW_b_ref[i][None, :]
        K1 = _mmT(ug, gf1_ref[i])
        K2 = _mmT(ug, gf2_ref[i])
        K3 = _mmT(ctx, gf3_ref[i]) + gf3_b_ref[i][None, :]
        bf16 = jnp.bfloat16
        u = (K1.reshape(_S, _G, _D) + K3[:, None, :]).astype(bf16)
        K2b = K2.reshape(_S, _G, _D).astype(bf16)
        gab = ga_ref[i].astype(bf16)
        chunks = []
        for ic in range(0, _G, 8):
            e3 = jax.nn.relu(u[:, None, :, :]
                             + K2b[:, ic:ic + 8, None, :])
            chunks.append(jnp.sum(e3 * gab[None, None, None, :], axis=-1,
                                  dtype=jnp.float32))
        a = jnp.concatenate(chunks, axis=1)
        e = jnp.where(a >= 0, a, 0.2 * a)
        e = jnp.where(adj == 0, -1e9, e)
        e = e - jnp.max(e, axis=-1, keepdims=True)
        p = jnp.exp(e)
        alpha = p / jnp.sum(p, axis=-1, keepdims=True)
        comb = _bmm(alpha, h.reshape(_S, _G, _D)).reshape(_S * _G, _D)
        ug = jax.nn.relu(comb) + ug

    ug3 = ug.reshape(_S, _G, _D)
    hist = ug3[:, :_H, :]
    Kh = _mmT(hist.reshape(_S * _H, _D), K_w_ref[...]).reshape(_S, _H, _D)
    Qc = _mmT(ctx, Q_w_ref[...]) + Q_b_ref[...]
    a_r = _bmv(Kh, Qc) * _INV_SQRT_D

    idx = idx_ref[...].reshape(_S, 1, _H)
    cat_iota = jax.lax.broadcasted_iota(jnp.int32, (_S, _C, _H), 1)
    M = idx == cat_iota
    neg_inf = jnp.float32(-jnp.inf)
    a_b = a_r[:, None, :]
    mxc = jnp.max(jnp.where(M, a_b, neg_inf), axis=2, keepdims=True)
    mxc = jnp.where(mxc == neg_inf, 0.0, mxc)
    m = jnp.sum(jnp.where(M, mxc, 0.0), axis=1, keepdims=True)
    ex = jnp.exp(a_b - m)
    den = jnp.sum(jnp.where(M, ex, 0.0), axis=2, keepdims=True)
    denh = jnp.sum(jnp.where(M, den, 0.0), axis=1, keepdims=True)
    w_sc = jnp.where(M, ex / denh, 0.0)
    topics = _bmm(w_sc, hist)
    t2 = jax.nn.relu(
        _mmT(topics.reshape(_S * _C, _D), fa_w_ref[...]) + fa_b_ref[...]
    ).reshape(_S, _C, _D) + topics

    Ks = _mmT(t2.reshape(_S * _C, _D), saK_ref[...]).reshape(_S, _C, _D)
    Qs = _mmT(ctx, saQ_ref[...]) + saQb_ref[...]
    s = _bmv(Ks, Qs) * _INV_SQRT_D
    cm = cmask_ref[...].reshape(_S, _C)
    s = jnp.where(cm == 0, -1e9, s)
    s = s - jnp.max(s, axis=-1, keepdims=True)
    ps = jnp.exp(s)
    al = ps / jnp.sum(ps, axis=-1, keepdims=True)
    out_ref[...] = _bmm(al[:, None, :], t2)


def kernel(news_graph_embeddings, news_graph, news_graph_mask,
           user_news_embedding, user_graph, user_category_mask,
           user_category_indices, topic_node_embedding,
           K_w, Q_w, Q_b, fa_w, fa_b, sa_K_w, sa_Q_w, sa_Q_b,
           gW_w, gW_b, gf1_w, gf2_w, gf3_w, gf3_b, ga_w):
    B = news_graph_embeddings.shape[0]
    f32 = jnp.float32

    ctx = news_graph_embeddings[:, 0, :]
    topic_nodes = jnp.broadcast_to(topic_node_embedding[None], (B, _CAT, _D))
    ug0 = jnp.concatenate([user_news_embedding, topic_nodes], axis=1)

    ctx3 = ctx.reshape(B, 1, _D)
    idx3 = user_category_indices.reshape(B, 1, _H).astype(jnp.int32)
    cmask3 = user_category_mask.reshape(B, 1, _C).astype(jnp.int32)
    Q_b2 = Q_b.reshape(1, _D)
    fa_b2 = fa_b.reshape(1, _D)
    sa_Q_b2 = sa_Q_b.reshape(1, _D)

    def full(spec_shape):
        return pl.BlockSpec(spec_shape, lambda b: (0,) * len(spec_shape))

    def per_b(spec_shape):
        return pl.BlockSpec(spec_shape, lambda b: (b,) + (0,) * (len(spec_shape) - 1))

    out = pl.pallas_call(
        _fused_kernel,
        grid=(B // _S,),
        in_specs=[
            per_b((_S, _G, _D)),
            per_b((_S, 1, _D)),
            per_b((_S, _G, _G)),
            per_b((_S, 1, _H)),
            per_b((_S, 1, _C)),
            full((_L, _D, _D)),
            full((_L, _D)),
            full((_L, _D, _D)),
            full((_L, _D, _D)),
            full((_L, _D, _D)),
            full((_L, _D)),
            full((_L, _D)),
            full((_D, _D)),
            full((_D, _D)),
            full((1, _D)),
            full((_D, _D)),
            full((1, _D)),
            full((_D, _D)),
            full((_D, _D)),
            full((1, _D)),
        ],
        out_specs=per_b((_S, 1, _D)),
        out_shape=jax.ShapeDtypeStruct((B, 1, _D), f32),
    )(ug0, ctx3, user_graph, idx3, cmask3,
      gW_w, gW_b, gf1_w, gf2_w, gf3_w, gf3_b, ga_w,
      K_w, Q_w, Q_b2, fa_w, fa_b2, sa_K_w, sa_Q_w, sa_Q_b2)

    return (ctx, out.reshape(B, _D))


# speedup vs baseline: 2.9766x; 1.0154x over previous
import jax
import jax.numpy as jnp
from jax.experimental import pallas as pl

_D = 256
_H = 50
_CAT = 30
_C = _CAT + 1
_G = _H + _CAT
_L = 2
_S = 8
_INV_SQRT_D = 0.0625


def _mmT(x, w):
    return jax.lax.dot_general(
        x, w, (((1,), (1,)), ((), ())), preferred_element_type=jnp.float32)


def _bmm(x, y):
    return jax.lax.dot_general(
        x, y, (((2,), (1,)), ((0,), (0,))), preferred_element_type=jnp.float32)


def _bmv(x, v):
    return jax.lax.dot_general(
        x, v, (((2,), (1,)), ((0,), (0,))), preferred_element_type=jnp.float32)


def _fused_kernel(une_ref, top_ref, ctx_ref, adj_ref, idx_ref, cmask_ref,
                  gW_w_ref, gW_b_ref, gf1_ref, gf2_ref, gf3_ref, gf3_b_ref,
                  ga_ref, K_w_ref, Q_w_ref, Q_b_ref, fa_w_ref, fa_b_ref,
                  saK_ref, saQ_ref, saQb_ref, out_ref):
    topics0 = jnp.broadcast_to(top_ref[...][None], (_S, _CAT, _D))
    ug = jnp.concatenate([une_ref[...], topics0],
                         axis=1).reshape(_S * _G, _D)
    ctx = ctx_ref[...].reshape(_S, _D)
    adj = adj_ref[...]

    for i in range(_L):
        h = _mmT(ug, gW_w_ref[i]) + gW_b_ref[i][None, :]
        K1 = _mmT(ug, gf1_ref[i])
        K2 = _mmT(ug, gf2_ref[i])
        K3 = _mmT(ctx, gf3_ref[i]) + gf3_b_ref[i][None, :]
        bf16 = jnp.bfloat16
        u = (K1.reshape(_S, _G, _D) + K3[:, None, :]).astype(bf16)
        K2b = K2.reshape(_S, _G, _D).astype(bf16)
        gab = ga_ref[i].astype(bf16)
        chunks = []
        for ic in range(0, _G, 8):
            e3 = jax.nn.relu(u[:, None, :, :]
                             + K2b[:, ic:ic + 8, None, :])
            chunks.append(jnp.sum(e3 * gab[None, None, None, :], axis=-1,
                                  dtype=jnp.float32))
        a = jnp.concatenate(chunks, axis=1)
        e = jnp.where(a >= 0, a, 0.2 * a)
        e = jnp.where(adj == 0, -1e9, e)
        e = e - jnp.max(e, axis=-1, keepdims=True)
        p = jnp.exp(e)
        alpha = p / jnp.sum(p, axis=-1, keepdims=True)
        comb = _bmm(alpha, h.reshape(_S, _G, _D)).reshape(_S * _G, _D)
        ug = jax.nn.relu(comb) + ug

    ug3 = ug.reshape(_S, _G, _D)
    hist = ug3[:, :_H, :]
    Kh = _mmT(hist.reshape(_S * _H, _D), K_w_ref[...]).reshape(_S, _H, _D)
    Qc = _mmT(ctx, Q_w_ref[...]) + Q_b_ref[...]
    a_r = _bmv(Kh, Qc) * _INV_SQRT_D

    idx = idx_ref[...].reshape(_S, 1, _H)
    cat_iota = jax.lax.broadcasted_iota(jnp.int32, (_S, _C, _H), 1)
    M = idx == cat_iota
    neg_inf = jnp.float32(-jnp.inf)
    a_b = a_r[:, None, :]
    mxc = jnp.max(jnp.where(M, a_b, neg_inf), axis=2, keepdims=True)
    mxc = jnp.where(mxc == neg_inf, 0.0, mxc)
    m = jnp.sum(jnp.where(M, mxc, 0.0), axis=1, keepdims=True)
    ex = jnp.exp(a_b - m)
    den = jnp.sum(jnp.where(M, ex, 0.0), axis=2, keepdims=True)
    denh = jnp.sum(jnp.where(M, den, 0.0), axis=1, keepdims=True)
    w_sc = jnp.where(M, ex / denh, 0.0)
    topics = _bmm(w_sc, hist)
    t2 = jax.nn.relu(
        _mmT(topics.reshape(_S * _C, _D), fa_w_ref[...]) + fa_b_ref[...]
    ).reshape(_S, _C, _D) + topics

    Ks = _mmT(t2.reshape(_S * _C, _D), saK_ref[...]).reshape(_S, _C, _D)
    Qs = _mmT(ctx, saQ_ref[...]) + saQb_ref[...]
    s = _bmv(Ks, Qs) * _INV_SQRT_D
    cm = cmask_ref[...].reshape(_S, _C)
    s = jnp.where(cm == 0, -1e9, s)
    s = s - jnp.max(s, axis=-1, keepdims=True)
    ps = jnp.exp(s)
    al = ps / jnp.sum(ps, axis=-1, keepdims=True)
    out_ref[...] = _bmm(al[:, None, :], t2)


def kernel(news_graph_embeddings, news_graph, news_graph_mask,
           user_news_embedding, user_graph, user_category_mask,
           user_category_indices, topic_node_embedding,
           K_w, Q_w, Q_b, fa_w, fa_b, sa_K_w, sa_Q_w, sa_Q_b,
           gW_w, gW_b, gf1_w, gf2_w, gf3_w, gf3_b, ga_w):
    B = news_graph_embeddings.shape[0]
    f32 = jnp.float32

    ctx = news_graph_embeddings[:, 0, :]
    ctx3 = ctx.reshape(B, 1, _D)
    idx3 = user_category_indices.reshape(B, 1, _H).astype(jnp.int32)
    cmask3 = user_category_mask.reshape(B, 1, _C).astype(jnp.int32)
    Q_b2 = Q_b.reshape(1, _D)
    fa_b2 = fa_b.reshape(1, _D)
    sa_Q_b2 = sa_Q_b.reshape(1, _D)

    def full(spec_shape):
        return pl.BlockSpec(spec_shape, lambda b: (0,) * len(spec_shape))

    def per_b(spec_shape):
        return pl.BlockSpec(spec_shape, lambda b: (b,) + (0,) * (len(spec_shape) - 1))

    out = pl.pallas_call(
        _fused_kernel,
        grid=(B // _S,),
        in_specs=[
            per_b((_S, _H, _D)),
            full((_CAT, _D)),
            per_b((_S, 1, _D)),
            per_b((_S, _G, _G)),
            per_b((_S, 1, _H)),
            per_b((_S, 1, _C)),
            full((_L, _D, _D)),
            full((_L, _D)),
            full((_L, _D, _D)),
            full((_L, _D, _D)),
            full((_L, _D, _D)),
            full((_L, _D)),
            full((_L, _D)),
            full((_D, _D)),
            full((_D, _D)),
            full((1, _D)),
            full((_D, _D)),
            full((1, _D)),
            full((_D, _D)),
            full((_D, _D)),
            full((1, _D)),
        ],
        out_specs=per_b((_S, 1, _D)),
        out_shape=jax.ShapeDtypeStruct((B, 1, _D), f32),
    )(user_news_embedding, topic_node_embedding, ctx3, user_graph, idx3, cmask3,
      gW_w, gW_b, gf1_w, gf2_w, gf3_w, gf3_b, ga_w,
      K_w, Q_w, Q_b2, fa_w, fa_b2, sa_K_w, sa_Q_w, sa_Q_b2)

    return (ctx, out.reshape(B, _D))
